# baseline (speedup 1.0000x reference)
import functools

import jax
import jax.numpy as jnp
from jax import lax
from jax.experimental import pallas as pl
from jax.experimental.pallas import tpu as pltpu
from jax.experimental.pallas import tpu_sc as plsc

MAX_PATH_LEN = 5
EDGE_FEAT_DIM = 128
NUM_HEADS = 32
N_NODES = 256
N_EDGES = 4096
B = N_NODES * N_NODES
P = 4224
PR = P * NUM_HEADS // 128
ER = N_EDGES * NUM_HEADS // 128

NUM_CORES = 2
NUM_SUBCORES = 16
NW = NUM_CORES * NUM_SUBCORES
PAIRS_PER_W = B // NW
CHUNK = 128
NCHUNK = PAIRS_PER_W // CHUNK
OUT_ROWS = CHUNK * NUM_HEADS // 128


def _proj_body(ef4_ref, wblk_ref, out_ref):
    mm = lax.dot_general(ef4_ref[...], wblk_ref[0],
                         (((1,), (0,)), ((), ())),
                         preferred_element_type=jnp.float32)
    out_ref[0:ER, :] = mm
    out_ref[ER:PR, :] = jnp.zeros((PR - ER, 128), jnp.float32)


def _build_table(ef4, wblk):
    return pl.pallas_call(
        _proj_body,
        grid=(MAX_PATH_LEN,),
        in_specs=[
            pl.BlockSpec((ER, 4 * EDGE_FEAT_DIM), lambda l: (0, 0)),
            pl.BlockSpec((1, 4 * EDGE_FEAT_DIM, 128), lambda l: (l, 0, 0)),
        ],
        out_specs=pl.BlockSpec((PR, 128), lambda l: (l, 0)),
        out_shape=jax.ShapeDtypeStruct((MAX_PATH_LEN * PR, 128), jnp.float32),
    )(ef4, wblk)


@functools.partial(
    pl.kernel,
    out_type=jax.ShapeDtypeStruct((B * NUM_HEADS // 128, 128), jnp.float32),
    mesh=plsc.VectorSubcoreMesh(core_axis_name="c", subcore_axis_name="s"),
    compiler_params=pltpu.CompilerParams(use_tc_tiling_on_sc=False),
    scratch_types=[
        pltpu.VMEM((NCHUNK * MAX_PATH_LEN * CHUNK,), jnp.int32),
        pltpu.VMEM((MAX_PATH_LEN * CHUNK, NUM_HEADS), jnp.float32),
        pltpu.VMEM((MAX_PATH_LEN * CHUNK, NUM_HEADS), jnp.float32),
        pltpu.VMEM((OUT_ROWS, 128), jnp.float32),
        pltpu.VMEM((OUT_ROWS, 128), jnp.float32),
        pltpu.VMEM_SHARED((MAX_PATH_LEN * P, NUM_HEADS), jnp.float32),
        pltpu.SemaphoreType.DMA,
        pltpu.SemaphoreType.DMA,
        pltpu.SemaphoreType.DMA,
        pltpu.SemaphoreType.DMA,
    ],
)
def _gather_accum(table_hbm, idx_hbm, out_hbm, idx_v,
                  rows_a, rows_b, out_a, out_b, table_sp,
                  sem_a, sem_b, sem_oa, sem_ob):
    wid = lax.axis_index("s") * NUM_CORES + lax.axis_index("c")
    nidx = NCHUNK * MAX_PATH_LEN * CHUNK
    pltpu.sync_copy(idx_hbm.at[pl.ds(wid * nidx, nidx)], idx_v)
    sid = lax.axis_index("s")
    srows = MAX_PATH_LEN * P // NUM_SUBCORES
    pltpu.sync_copy(table_hbm.at[pl.ds(sid * srows, srows)],
                    table_sp.at[pl.ds(sid * srows, srows)])
    plsc.subcore_barrier()
    out_base = wid * (PAIRS_PER_W * NUM_HEADS // 128)
    cidx = MAX_PATH_LEN * CHUNK

    def issue(c, rows_v, sem):
        pltpu.async_copy(table_sp.at[idx_v.at[pl.ds(c * cidx, cidx)]],
                         rows_v, sem)

    def drain(rows_v, sem):
        pltpu.make_async_copy(
            table_hbm.at[pl.ds(0, MAX_PATH_LEN * CHUNK)], rows_v, sem).wait()

    def drain_out(out_v, sem_o):
        pltpu.make_async_copy(
            out_v, out_hbm.at[pl.ds(out_base, OUT_ROWS)], sem_o).wait()

    def accum(c, rows_v, out_v, sem_o, not_first):
        @pl.when(not_first)
        def _():
            drain_out(out_v, sem_o)

        @plsc.parallel_loop(0, OUT_ROWS, unroll=2)
        def _(row):
            j0 = row * (4 * MAX_PATH_LEN)
            for k in range(4):
                for h in range(NUM_HEADS // 16):
                    sl = pl.ds(h * 16, 16)
                    acc = rows_v[j0 + 5 * k, sl]
                    for l in range(1, MAX_PATH_LEN):
                        acc = acc + rows_v[j0 + 5 * k + l, sl]
                    out_v[row, pl.ds(k * 32 + h * 16, 16)] = acc

        pltpu.async_copy(out_v, out_hbm.at[pl.ds(out_base + c * OUT_ROWS,
                                                 OUT_ROWS)], sem_o)

    issue(0, rows_a, sem_a)

    def body(t, carry):
        c0 = 2 * t
        issue(c0 + 1, rows_b, sem_b)
        drain(rows_a, sem_a)
        accum(c0, rows_a, out_a, sem_oa, t > 0)

        @pl.when(t < NCHUNK // 2 - 1)
        def _():
            issue(c0 + 2, rows_a, sem_a)

        drain(rows_b, sem_b)
        accum(c0 + 1, rows_b, out_b, sem_ob, t > 0)
        return carry

    lax.fori_loop(0, NCHUNK // 2, body, 0)
    drain_out(out_a, sem_oa)
    drain_out(out_b, sem_ob)


def kernel(shortest_paths, edge_feat, max_shortest_path_len, weight_embedding):
    mask = (jnp.arange(MAX_PATH_LEN)
            < jnp.minimum(MAX_PATH_LEN, max_shortest_path_len))
    w = weight_embedding[:MAX_PATH_LEN * NUM_HEADS].reshape(
        MAX_PATH_LEN, NUM_HEADS, EDGE_FEAT_DIM)
    w = w * mask.astype(w.dtype)[:, None, None]
    wblk = jax.vmap(
        lambda m: jnp.kron(jnp.eye(4, dtype=m.dtype), m.T))(w)
    ef4 = edge_feat.reshape(ER, 4 * EDGE_FEAT_DIM)

    table = _build_table(ef4, wblk).reshape(MAX_PATH_LEN * P, NUM_HEADS)

    sp = shortest_paths.reshape(B, MAX_PATH_LEN).astype(jnp.int32)
    idx = sp + (jnp.arange(MAX_PATH_LEN, dtype=jnp.int32) * P)[None, :]
    idx = idx.reshape(B * MAX_PATH_LEN)

    out = _gather_accum(table, idx)
    return out.reshape(N_NODES, N_NODES, NUM_HEADS)


# speedup vs baseline: 1.8891x; 1.8891x over previous
import functools

import jax
import jax.numpy as jnp
from jax import lax
from jax.experimental import pallas as pl
from jax.experimental.pallas import tpu as pltpu
from jax.experimental.pallas import tpu_sc as plsc

MAX_PATH_LEN = 5
EDGE_FEAT_DIM = 128
NUM_HEADS = 32
N_NODES = 256
N_EDGES = 4096
B = N_NODES * N_NODES
P = 4224
PR = P * NUM_HEADS // 128
ER = N_EDGES * NUM_HEADS // 128

NUM_CORES = 2
NUM_SUBCORES = 16
NW = NUM_CORES * NUM_SUBCORES
PAIRS_PER_W = B // NW
CHUNK = 128
NCHUNK = PAIRS_PER_W // CHUNK
OUT_ROWS = CHUNK * NUM_HEADS // 128


def _proj_body(ef4_ref, wblk_ref, out_ref):
    mm = lax.dot_general(ef4_ref[...], wblk_ref[0],
                         (((1,), (0,)), ((), ())),
                         preferred_element_type=jnp.float32)
    out_ref[0:ER, :] = mm
    out_ref[ER:PR, :] = jnp.zeros((PR - ER, 128), jnp.float32)


def _build_table(ef4, wblk):
    return pl.pallas_call(
        _proj_body,
        grid=(MAX_PATH_LEN,),
        in_specs=[
            pl.BlockSpec((ER, 4 * EDGE_FEAT_DIM), lambda l: (0, 0)),
            pl.BlockSpec((1, 4 * EDGE_FEAT_DIM, 128), lambda l: (l, 0, 0)),
        ],
        out_specs=pl.BlockSpec((PR, 128), lambda l: (l, 0)),
        out_shape=jax.ShapeDtypeStruct((MAX_PATH_LEN * PR, 128), jnp.float32),
    )(ef4, wblk)


@functools.partial(
    pl.kernel,
    out_type=jax.ShapeDtypeStruct((B * NUM_HEADS // 128, 128), jnp.float32),
    mesh=plsc.VectorSubcoreMesh(core_axis_name="c", subcore_axis_name="s"),
    compiler_params=pltpu.CompilerParams(use_tc_tiling_on_sc=False),
    scratch_types=[
        pltpu.VMEM((NCHUNK * MAX_PATH_LEN * CHUNK,), jnp.int32),
        pltpu.VMEM((MAX_PATH_LEN * CHUNK, NUM_HEADS), jnp.float32),
        pltpu.VMEM((MAX_PATH_LEN * CHUNK, NUM_HEADS), jnp.float32),
        pltpu.VMEM((OUT_ROWS, 128), jnp.float32),
        pltpu.VMEM((OUT_ROWS, 128), jnp.float32),
        pltpu.VMEM_SHARED((MAX_PATH_LEN * P, NUM_HEADS), jnp.float32),
        pltpu.SemaphoreType.DMA,
        pltpu.SemaphoreType.DMA,
        pltpu.SemaphoreType.DMA,
        pltpu.SemaphoreType.DMA,
    ],
)
def _gather_accum(table_hbm, idx_hbm, out_hbm, idx_v,
                  rows_a, rows_b, out_a, out_b, table_sp,
                  sem_a, sem_b, sem_oa, sem_ob):
    wid = lax.axis_index("s") * NUM_CORES + lax.axis_index("c")
    nidx = NCHUNK * MAX_PATH_LEN * CHUNK
    pltpu.sync_copy(idx_hbm.at[pl.ds(wid * nidx, nidx)], idx_v)
    sid = lax.axis_index("s")
    srows = MAX_PATH_LEN * P // NUM_SUBCORES
    pltpu.sync_copy(table_hbm.at[pl.ds(sid * srows, srows)],
                    table_sp.at[pl.ds(sid * srows, srows)])
    plsc.subcore_barrier()
    out_base = wid * (PAIRS_PER_W * NUM_HEADS // 128)
    cidx = MAX_PATH_LEN * CHUNK

    def issue(c, rows_v, sem):
        pltpu.async_copy(table_sp.at[idx_v.at[pl.ds(c * cidx, cidx)]],
                         rows_v, sem)

    def drain(rows_v, sem):
        pltpu.make_async_copy(
            table_hbm.at[pl.ds(0, MAX_PATH_LEN * CHUNK)], rows_v, sem).wait()

    def drain_out(out_v, sem_o):
        pltpu.make_async_copy(
            out_v, out_hbm.at[pl.ds(out_base, OUT_ROWS)], sem_o).wait()

    def accum(c, rows_v, out_v, sem_o, not_first):
        @pl.when(not_first)
        def _():
            drain_out(out_v, sem_o)

        @plsc.parallel_loop(0, OUT_ROWS, unroll=2)
        def _(row):
            j0 = lax.shift_left(row, 2)
            for k in range(4):
                for h in range(NUM_HEADS // 16):
                    sl = pl.ds(h * 16, 16)
                    acc = rows_v[j0 + k, sl]
                    for l in range(1, MAX_PATH_LEN):
                        acc = acc + rows_v[l * CHUNK + j0 + k, sl]
                    out_v[row, pl.ds(k * 32 + h * 16, 16)] = acc

        pltpu.async_copy(out_v, out_hbm.at[pl.ds(out_base + c * OUT_ROWS,
                                                 OUT_ROWS)], sem_o)

    issue(0, rows_a, sem_a)

    def body(t, carry):
        c0 = 2 * t
        issue(c0 + 1, rows_b, sem_b)
        drain(rows_a, sem_a)
        accum(c0, rows_a, out_a, sem_oa, t > 0)

        @pl.when(t < NCHUNK // 2 - 1)
        def _():
            issue(c0 + 2, rows_a, sem_a)

        drain(rows_b, sem_b)
        accum(c0 + 1, rows_b, out_b, sem_ob, t > 0)
        return carry

    lax.fori_loop(0, NCHUNK // 2, body, 0)
    drain_out(out_a, sem_oa)
    drain_out(out_b, sem_ob)


def kernel(shortest_paths, edge_feat, max_shortest_path_len, weight_embedding):
    mask = (jnp.arange(MAX_PATH_LEN)
            < jnp.minimum(MAX_PATH_LEN, max_shortest_path_len))
    w = weight_embedding[:MAX_PATH_LEN * NUM_HEADS].reshape(
        MAX_PATH_LEN, NUM_HEADS, EDGE_FEAT_DIM)
    w = w * mask.astype(w.dtype)[:, None, None]
    wblk = jax.vmap(
        lambda m: jnp.kron(jnp.eye(4, dtype=m.dtype), m.T))(w)
    ef4 = edge_feat.reshape(ER, 4 * EDGE_FEAT_DIM)

    table = _build_table(ef4, wblk).reshape(MAX_PATH_LEN * P, NUM_HEADS)

    sp = shortest_paths.reshape(B, MAX_PATH_LEN).astype(jnp.int32)
    idx = sp + (jnp.arange(MAX_PATH_LEN, dtype=jnp.int32) * P)[None, :]
    idx = (idx.reshape(NW, NCHUNK, CHUNK, MAX_PATH_LEN)
              .transpose(0, 1, 3, 2)
              .reshape(NW * NCHUNK * MAX_PATH_LEN * CHUNK))

    out = _gather_accum(table, idx)
    return out.reshape(N_NODES, N_NODES, NUM_HEADS)


# speedup vs baseline: 1.8956x; 1.0035x over previous
import functools

import jax
import jax.numpy as jnp
from jax import lax
from jax.experimental import pallas as pl
from jax.experimental.pallas import tpu as pltpu
from jax.experimental.pallas import tpu_sc as plsc

MAX_PATH_LEN = 5
EDGE_FEAT_DIM = 128
NUM_HEADS = 32
N_NODES = 256
N_EDGES = 4096
B = N_NODES * N_NODES
P = 4224
PR = P * NUM_HEADS // 128
ER = N_EDGES * NUM_HEADS // 128

NUM_CORES = 2
NUM_SUBCORES = 16
NW = NUM_CORES * NUM_SUBCORES
PAIRS_PER_W = B // NW
CHUNK = 128
NCHUNK = PAIRS_PER_W // CHUNK
OUT_ROWS = CHUNK * NUM_HEADS // 128


def _proj_body(ef4_ref, wblk_ref, out_ref):
    mm = lax.dot_general(ef4_ref[...], wblk_ref[0],
                         (((1,), (0,)), ((), ())),
                         preferred_element_type=jnp.float32)
    out_ref[0:ER, :] = mm
    out_ref[ER:PR, :] = jnp.zeros((PR - ER, 128), jnp.float32)


def _build_table(ef4, wblk):
    return pl.pallas_call(
        _proj_body,
        grid=(MAX_PATH_LEN,),
        in_specs=[
            pl.BlockSpec((ER, 4 * EDGE_FEAT_DIM), lambda l: (0, 0)),
            pl.BlockSpec((1, 4 * EDGE_FEAT_DIM, 128), lambda l: (l, 0, 0)),
        ],
        out_specs=pl.BlockSpec((PR, 128), lambda l: (l, 0)),
        out_shape=jax.ShapeDtypeStruct((MAX_PATH_LEN * PR, 128), jnp.float32),
    )(ef4, wblk)


@functools.partial(
    pl.kernel,
    out_type=jax.ShapeDtypeStruct((B * NUM_HEADS // 128, 128), jnp.float32),
    mesh=plsc.VectorSubcoreMesh(core_axis_name="c", subcore_axis_name="s"),
    compiler_params=pltpu.CompilerParams(use_tc_tiling_on_sc=False),
    scratch_types=[
        pltpu.VMEM((NCHUNK * MAX_PATH_LEN * CHUNK,), jnp.int32),
        pltpu.VMEM((MAX_PATH_LEN * CHUNK, NUM_HEADS), jnp.float32),
        pltpu.VMEM((MAX_PATH_LEN * CHUNK, NUM_HEADS), jnp.float32),
        pltpu.VMEM((OUT_ROWS, 128), jnp.float32),
        pltpu.VMEM((OUT_ROWS, 128), jnp.float32),
        pltpu.VMEM_SHARED((MAX_PATH_LEN * P, NUM_HEADS), jnp.float32),
        pltpu.SemaphoreType.DMA,
        pltpu.SemaphoreType.DMA,
        pltpu.SemaphoreType.DMA,
        pltpu.SemaphoreType.DMA,
    ],
)
def _gather_accum(table_hbm, idx_hbm, out_hbm, idx_v,
                  rows_a, rows_b, out_a, out_b, table_sp,
                  sem_a, sem_b, sem_oa, sem_ob):
    wid = lax.axis_index("s") * NUM_CORES + lax.axis_index("c")
    nidx = NCHUNK * MAX_PATH_LEN * CHUNK
    pltpu.sync_copy(idx_hbm.at[pl.ds(wid * nidx, nidx)], idx_v)
    sid = lax.axis_index("s")
    srows = MAX_PATH_LEN * P // NUM_SUBCORES
    pltpu.sync_copy(table_hbm.at[pl.ds(sid * srows, srows)],
                    table_sp.at[pl.ds(sid * srows, srows)])
    plsc.subcore_barrier()
    out_base = wid * (PAIRS_PER_W * NUM_HEADS // 128)
    cidx = MAX_PATH_LEN * CHUNK

    def issue(c, rows_v, sem):
        pltpu.async_copy(table_sp.at[idx_v.at[pl.ds(c * cidx, cidx)]],
                         rows_v, sem)

    def drain(rows_v, sem):
        pltpu.make_async_copy(
            table_hbm.at[pl.ds(0, MAX_PATH_LEN * CHUNK)], rows_v, sem).wait()

    def drain_out(out_v, sem_o):
        pltpu.make_async_copy(
            out_v, out_hbm.at[pl.ds(out_base, OUT_ROWS)], sem_o).wait()

    def accum(c, rows_v, out_v, sem_o, not_first):
        @pl.when(not_first)
        def _():
            drain_out(out_v, sem_o)

        @plsc.parallel_loop(0, OUT_ROWS, unroll=4)
        def _(row):
            j0 = lax.shift_left(row, 2)
            for k in range(4):
                for h in range(NUM_HEADS // 16):
                    sl = pl.ds(h * 16, 16)
                    acc = rows_v[j0 + k, sl]
                    for l in range(1, MAX_PATH_LEN):
                        acc = acc + rows_v[l * CHUNK + j0 + k, sl]
                    out_v[row, pl.ds(k * 32 + h * 16, 16)] = acc

        pltpu.async_copy(out_v, out_hbm.at[pl.ds(out_base + c * OUT_ROWS,
                                                 OUT_ROWS)], sem_o)

    issue(0, rows_a, sem_a)

    def body(t, carry):
        c0 = 2 * t
        issue(c0 + 1, rows_b, sem_b)
        drain(rows_a, sem_a)
        accum(c0, rows_a, out_a, sem_oa, t > 0)

        @pl.when(t < NCHUNK // 2 - 1)
        def _():
            issue(c0 + 2, rows_a, sem_a)

        drain(rows_b, sem_b)
        accum(c0 + 1, rows_b, out_b, sem_ob, t > 0)
        return carry

    lax.fori_loop(0, NCHUNK // 2, body, 0)
    drain_out(out_a, sem_oa)
    drain_out(out_b, sem_ob)


def kernel(shortest_paths, edge_feat, max_shortest_path_len, weight_embedding):
    mask = (jnp.arange(MAX_PATH_LEN)
            < jnp.minimum(MAX_PATH_LEN, max_shortest_path_len))
    w = weight_embedding[:MAX_PATH_LEN * NUM_HEADS].reshape(
        MAX_PATH_LEN, NUM_HEADS, EDGE_FEAT_DIM)
    w = w * mask.astype(w.dtype)[:, None, None]
    wblk = jax.vmap(
        lambda m: jnp.kron(jnp.eye(4, dtype=m.dtype), m.T))(w)
    ef4 = edge_feat.reshape(ER, 4 * EDGE_FEAT_DIM)

    table = _build_table(ef4, wblk).reshape(MAX_PATH_LEN * P, NUM_HEADS)

    sp = shortest_paths.reshape(B, MAX_PATH_LEN).astype(jnp.int32)
    idx = sp + (jnp.arange(MAX_PATH_LEN, dtype=jnp.int32) * P)[None, :]
    idx = (idx.reshape(NW, NCHUNK, CHUNK, MAX_PATH_LEN)
              .transpose(0, 1, 3, 2)
              .reshape(NW * NCHUNK * MAX_PATH_LEN * CHUNK))

    out = _gather_accum(table, idx)
    return out.reshape(N_NODES, N_NODES, NUM_HEADS)


# speedup vs baseline: 1.9263x; 1.0162x over previous
import functools

import jax
import jax.numpy as jnp
from jax import lax
from jax.experimental import pallas as pl
from jax.experimental.pallas import tpu as pltpu
from jax.experimental.pallas import tpu_sc as plsc

MAX_PATH_LEN = 5
EDGE_FEAT_DIM = 128
NUM_HEADS = 32
N_NODES = 256
N_EDGES = 4096
B = N_NODES * N_NODES
P = 4224
PR = P * NUM_HEADS // 128
ER = N_EDGES * NUM_HEADS // 128

NUM_CORES = 2
NUM_SUBCORES = 16
NW = NUM_CORES * NUM_SUBCORES
PAIRS_PER_W = B // NW
CHUNK = 128
NCHUNK = PAIRS_PER_W // CHUNK
OUT_ROWS = CHUNK * NUM_HEADS // 128


def _proj_body(ef4_ref, wblk_ref, out_ref):
    mm = lax.dot_general(ef4_ref[...], wblk_ref[0],
                         (((1,), (0,)), ((), ())),
                         preferred_element_type=jnp.float32)
    out_ref[0:ER, :] = mm
    out_ref[ER:PR, :] = jnp.zeros((PR - ER, 128), jnp.float32)


def _build_table(ef4, wblk):
    return pl.pallas_call(
        _proj_body,
        grid=(MAX_PATH_LEN,),
        in_specs=[
            pl.BlockSpec((ER, 4 * EDGE_FEAT_DIM), lambda l: (0, 0)),
            pl.BlockSpec((1, 4 * EDGE_FEAT_DIM, 128), lambda l: (l, 0, 0)),
        ],
        out_specs=pl.BlockSpec((PR, 128), lambda l: (l, 0)),
        out_shape=jax.ShapeDtypeStruct((MAX_PATH_LEN * PR, 128), jnp.float32),
    )(ef4, wblk)


@functools.partial(
    pl.kernel,
    out_type=jax.ShapeDtypeStruct((B * NUM_HEADS // 128, 128), jnp.float32),
    mesh=plsc.VectorSubcoreMesh(core_axis_name="c", subcore_axis_name="s"),
    compiler_params=pltpu.CompilerParams(use_tc_tiling_on_sc=False),
    scratch_types=[
        pltpu.VMEM((NCHUNK * MAX_PATH_LEN * CHUNK,), jnp.int32),
        pltpu.VMEM((MAX_PATH_LEN * CHUNK, NUM_HEADS), jnp.float32),
        pltpu.VMEM((MAX_PATH_LEN * CHUNK, NUM_HEADS), jnp.float32),
        pltpu.VMEM((OUT_ROWS, 128), jnp.float32),
        pltpu.VMEM((OUT_ROWS, 128), jnp.float32),
        pltpu.VMEM_SHARED((MAX_PATH_LEN * P, NUM_HEADS), jnp.float32),
        pltpu.SemaphoreType.DMA,
        pltpu.SemaphoreType.DMA,
        pltpu.SemaphoreType.DMA,
        pltpu.SemaphoreType.DMA,
    ],
)
def _gather_accum(table_hbm, idx_hbm, out_hbm, idx_v,
                  rows_a, rows_b, out_a, out_b, table_sp,
                  sem_a, sem_b, sem_oa, sem_ob):
    wid = lax.axis_index("s") * NUM_CORES + lax.axis_index("c")
    nidx = NCHUNK * MAX_PATH_LEN * CHUNK
    pltpu.sync_copy(idx_hbm.at[pl.ds(wid * nidx, nidx)], idx_v)
    out_base = wid * (PAIRS_PER_W * NUM_HEADS // 128)
    cidx = MAX_PATH_LEN * CHUNK
    pltpu.async_copy(table_hbm.at[idx_v.at[pl.ds(0, cidx)]], rows_a, sem_a)
    sid = lax.axis_index("s")
    srows = MAX_PATH_LEN * P // NUM_SUBCORES
    pltpu.sync_copy(table_hbm.at[pl.ds(sid * srows, srows)],
                    table_sp.at[pl.ds(sid * srows, srows)])
    plsc.subcore_barrier()

    def issue(c, rows_v, sem):
        pltpu.async_copy(table_sp.at[idx_v.at[pl.ds(c * cidx, cidx)]],
                         rows_v, sem)

    def drain(rows_v, sem):
        pltpu.make_async_copy(
            table_hbm.at[pl.ds(0, MAX_PATH_LEN * CHUNK)], rows_v, sem).wait()

    def drain_out(out_v, sem_o):
        pltpu.make_async_copy(
            out_v, out_hbm.at[pl.ds(out_base, OUT_ROWS)], sem_o).wait()

    def accum(c, rows_v, out_v, sem_o, not_first):
        @pl.when(not_first)
        def _():
            drain_out(out_v, sem_o)

        @plsc.parallel_loop(0, OUT_ROWS, unroll=4)
        def _(row):
            j0 = lax.shift_left(row, 2)
            for k in range(4):
                for h in range(NUM_HEADS // 16):
                    sl = pl.ds(h * 16, 16)
                    acc = rows_v[j0 + k, sl]
                    for l in range(1, MAX_PATH_LEN):
                        acc = acc + rows_v[l * CHUNK + j0 + k, sl]
                    out_v[row, pl.ds(k * 32 + h * 16, 16)] = acc

        pltpu.async_copy(out_v, out_hbm.at[pl.ds(out_base + c * OUT_ROWS,
                                                 OUT_ROWS)], sem_o)

    def body(t, carry):
        c0 = 2 * t
        issue(c0 + 1, rows_b, sem_b)
        drain(rows_a, sem_a)
        accum(c0, rows_a, out_a, sem_oa, t > 0)

        @pl.when(t < NCHUNK // 2 - 1)
        def _():
            issue(c0 + 2, rows_a, sem_a)

        drain(rows_b, sem_b)
        accum(c0 + 1, rows_b, out_b, sem_ob, t > 0)
        return carry

    lax.fori_loop(0, NCHUNK // 2, body, 0)
    drain_out(out_a, sem_oa)
    drain_out(out_b, sem_ob)


def kernel(shortest_paths, edge_feat, max_shortest_path_len, weight_embedding):
    mask = (jnp.arange(MAX_PATH_LEN)
            < jnp.minimum(MAX_PATH_LEN, max_shortest_path_len))
    w = weight_embedding[:MAX_PATH_LEN * NUM_HEADS].reshape(
        MAX_PATH_LEN, NUM_HEADS, EDGE_FEAT_DIM)
    w = w * mask.astype(w.dtype)[:, None, None]
    wblk = jax.vmap(
        lambda m: jnp.kron(jnp.eye(4, dtype=m.dtype), m.T))(w)
    ef4 = edge_feat.reshape(ER, 4 * EDGE_FEAT_DIM)

    table = _build_table(ef4, wblk).reshape(MAX_PATH_LEN * P, NUM_HEADS)

    sp = shortest_paths.reshape(B, MAX_PATH_LEN).astype(jnp.int32)
    idx = sp + (jnp.arange(MAX_PATH_LEN, dtype=jnp.int32) * P)[None, :]
    idx = (idx.reshape(NW, NCHUNK, CHUNK, MAX_PATH_LEN)
              .transpose(0, 1, 3, 2)
              .reshape(NW * NCHUNK * MAX_PATH_LEN * CHUNK))

    out = _gather_accum(table, idx)
    return out.reshape(N_NODES, N_NODES, NUM_HEADS)
